# baseline (speedup 1.0000x reference)
import functools

import jax
import jax.numpy as jnp
from jax import lax
from jax.experimental import pallas as pl
from jax.experimental.pallas import tpu as pltpu
from jax.experimental.pallas import tpu_sc as plsc

N = 10000
E = 320000
D = 128
H = 128
G = 64

NC = 2
NS = 16
NW = NC * NS

NP = 10112
CH = 128
EPW = 10240
NCHUNK = EPW // CH
EP = EPW * NW
RPS = NP // NS

_mesh = plsc.VectorSubcoreMesh(core_axis_name="c", subcore_axis_name="s")


@functools.partial(
    pl.kernel,
    out_type=jax.ShapeDtypeStruct((NC, NP), jnp.float32),
    mesh=_mesh,
    scratch_types=[
        pltpu.VMEM((NCHUNK, CH), jnp.int32),
        pltpu.VMEM((CH,), jnp.float32),
        pltpu.VMEM_SHARED((NP,), jnp.float32),
        pltpu.SemaphoreType.DMA,
    ],
)
def _deg_sc(col_hbm, ones_hbm, zeros_hbm, out_hbm, colbuf, ones_v, deg_sh, sem):
    cid = lax.axis_index("c")
    sid = lax.axis_index("s")
    wid = sid * NC + cid

    @pl.when(sid == 0)
    def _():
        pltpu.sync_copy(zeros_hbm, deg_sh)

    pltpu.sync_copy(ones_hbm, ones_v)
    pltpu.sync_copy(col_hbm.at[wid], colbuf)
    plsc.subcore_barrier()

    def body(ci, carry):
        pltpu.async_copy(ones_v, deg_sh.at[colbuf.at[ci]], sem, add=True).wait()
        return carry

    lax.fori_loop(0, NCHUNK, body, 0)
    plsc.subcore_barrier()

    @pl.when(sid == 0)
    def _():
        pltpu.sync_copy(deg_sh, out_hbm.at[cid])


@functools.partial(
    pl.kernel,
    out_type=jax.ShapeDtypeStruct((NC, NP, H), jnp.float32),
    mesh=_mesh,
    scratch_types=[
        pltpu.VMEM((4, 2, CH), jnp.int32),
        pltpu.VMEM((2, CH, H), jnp.float32),
        pltpu.VMEM_SHARED((NP, H), jnp.float32),
        pltpu.SemaphoreType.DMA,
        pltpu.SemaphoreType.DMA,
        pltpu.SemaphoreType.DMA,
        pltpu.SemaphoreType.DMA,
        pltpu.SemaphoreType.DMA,
        pltpu.SemaphoreType.DMA,
        pltpu.SemaphoreType.DMA,
        pltpu.SemaphoreType.DMA,
    ],
)
def _agg_sc(hs_hbm, rc_hbm, zeros_hbm, out_hbm,
            idx, rows, agg_sh, i0, i1, i2, i3, g0, g1, s0, s1):
    cid = lax.axis_index("c")
    sid = lax.axis_index("s")
    wid = sid * NC + cid
    isems = (i0, i1, i2, i3)
    gsems = (g0, g1)
    ssems = (s0, s1)

    def idx_start(c, s4):
        pltpu.async_copy(rc_hbm.at[wid, c], idx.at[s4], isems[s4])

    def idx_wait(s4):
        pltpu.make_async_copy(rc_hbm.at[wid, 0], idx.at[s4],
                              isems[s4]).wait()

    def gather_start(s4, s2):
        pltpu.async_copy(hs_hbm.at[idx.at[s4, 0]], rows.at[s2], gsems[s2])

    def gather_wait(s4, s2):
        pltpu.make_async_copy(hs_hbm.at[idx.at[s4, 0]], rows.at[s2],
                              gsems[s2]).wait()

    def scatter_start(s4, s2):
        pltpu.async_copy(rows.at[s2], agg_sh.at[idx.at[s4, 1]], ssems[s2],
                         add=True)

    def scatter_wait(s4, s2):
        pltpu.make_async_copy(rows.at[s2], agg_sh.at[idx.at[s4, 1]],
                              ssems[s2]).wait()

    pltpu.sync_copy(zeros_hbm.at[pl.ds(sid * RPS, RPS)],
                    agg_sh.at[pl.ds(sid * RPS, RPS)])
    idx_start(0, 0)
    idx_start(1, 1)
    plsc.subcore_barrier()
    idx_wait(0)
    gather_start(0, 0)

    def step(c, k):
        s4 = k
        s2 = k & 1

        @pl.when(c >= 1)
        def _():
            scatter_wait((k - 1) & 3, (k - 1) & 1)

        @pl.when(c + 1 < NCHUNK)
        def _():
            idx_wait((k + 1) & 3)
            gather_start((k + 1) & 3, (k + 1) & 1)

        gather_wait(s4, s2)
        scatter_start(s4, s2)

        @pl.when(c + 2 < NCHUNK)
        def _():
            idx_start(c + 2, (k + 2) & 3)

    def body(q, carry):
        for k in range(4):
            step(4 * q + k, k)
        return carry

    lax.fori_loop(0, NCHUNK // 4, body, 0)
    scatter_wait(3, 1)
    plsc.subcore_barrier()

    pltpu.sync_copy(agg_sh.at[pl.ds(sid * RPS, RPS)],
                    out_hbm.at[cid, pl.ds(sid * RPS, RPS)])


def _prep_body(deg_ref, x_ref, w_ref, dinv_ref, hs_ref):
    deg = deg_ref[0] + deg_ref[1] + 1.0
    dinv = lax.rsqrt(deg)[:, None]
    dinv_ref[...] = dinv
    hs_ref[...] = jnp.dot(x_ref[...], w_ref[...],
                          preferred_element_type=jnp.float32) * dinv


_prep_tc = pl.pallas_call(
    _prep_body,
    out_shape=(jax.ShapeDtypeStruct((NP, 1), jnp.float32),
               jax.ShapeDtypeStruct((NP, H), jnp.float32)),
)


def _mid_body(agg_ref, hs_ref, dinv_ref, b_ref, w_ref, out_ref):
    dinv = dinv_ref[...]
    t = (agg_ref[0] + agg_ref[1] + hs_ref[...]) * dinv + b_ref[...]
    t = jnp.maximum(t, 0.0)
    out_ref[...] = jnp.dot(t, w_ref[...],
                           preferred_element_type=jnp.float32) * dinv


_mid_tc = pl.pallas_call(
    _mid_body,
    out_shape=jax.ShapeDtypeStruct((NP, H), jnp.float32),
)


def _final_body(agg_ref, hs_ref, dinv_ref, b_ref, batch_ref, l1w_ref, l1b_ref,
                gam_ref, beta_ref, l2w_ref, l2b_ref, out_ref):
    h3 = (agg_ref[0] + agg_ref[1] + hs_ref[...]) * dinv_ref[...] + b_ref[...]
    onehot = (lax.broadcasted_iota(jnp.int32, (G, NP), 0)
              == batch_ref[...]).astype(jnp.float32)
    gp = jnp.dot(onehot, h3, preferred_element_type=jnp.float32,
                 precision=lax.Precision.HIGHEST)
    z = jnp.dot(gp, l1w_ref[...], preferred_element_type=jnp.float32)
    z = jnp.maximum(z + l1b_ref[...], 0.0)
    mean = jnp.mean(z, axis=0, keepdims=True)
    var = jnp.mean((z - mean) ** 2, axis=0, keepdims=True)
    z = (z - mean) / jnp.sqrt(var + 1e-5) * gam_ref[...] + beta_ref[...]
    out_ref[...] = jnp.dot(z, l2w_ref[...],
                           preferred_element_type=jnp.float32) + l2b_ref[...]


_final_tc = pl.pallas_call(
    _final_body,
    out_shape=jax.ShapeDtypeStruct((G, 1), jnp.float32),
)


def kernel(x, edge_index, batch, W0, b0, W1, b1, W2, b2,
           lin1_W, lin1_b, bn_gamma, bn_beta, lin2_W, lin2_b):
    pad_e = EP - E
    rowp = jnp.concatenate(
        [edge_index[0], jnp.zeros((pad_e,), jnp.int32)]).reshape(NW, NCHUNK, CH)
    colp = jnp.concatenate(
        [edge_index[1], jnp.full((pad_e,), N, jnp.int32)]).reshape(NW, NCHUNK, CH)
    rc = jnp.stack([rowp, colp], axis=2)
    xp = jnp.pad(x, ((0, NP - N), (0, 0)))
    batchp = jnp.pad(batch, (0, NP - N), constant_values=G).reshape(1, NP)
    zeros2d = jnp.zeros((NP, H), jnp.float32)
    zeros1d = jnp.zeros((NP,), jnp.float32)
    ones_ch = jnp.ones((CH,), jnp.float32)

    deg2 = _deg_sc(colp, ones_ch, zeros1d)
    dinv, hs = _prep_tc(deg2, xp, W0)
    agg = _agg_sc(hs, rc, zeros2d)
    hs = _mid_tc(agg, hs, dinv, b0.reshape(1, H), W1)
    agg = _agg_sc(hs, rc, zeros2d)
    hs = _mid_tc(agg, hs, dinv, b1.reshape(1, H), W2)
    agg = _agg_sc(hs, rc, zeros2d)
    return _final_tc(agg, hs, dinv, b2.reshape(1, H), batchp,
                     lin1_W, lin1_b.reshape(1, 64),
                     bn_gamma.reshape(1, 64), bn_beta.reshape(1, 64),
                     lin2_W, lin2_b.reshape(1, 1))


# speedup vs baseline: 1.1595x; 1.1595x over previous
import functools

import jax
import jax.numpy as jnp
from jax import lax
from jax.experimental import pallas as pl
from jax.experimental.pallas import tpu as pltpu
from jax.experimental.pallas import tpu_sc as plsc

N = 10000
E = 320000
D = 128
H = 128
G = 64

NC = 2
NS = 16
NW = NC * NS

NP = 10112
CH = 128
EPW = 10240
NCHUNK = EPW // CH
EP = EPW * NW
RPS = NP // NS

HALF = NP // 2
AGGH = 5120
RPSH = AGGH // NS
CH2 = 32
EPT = EP // NS
NCH2 = EPT // CH2
HSR = 10048

_mesh = plsc.VectorSubcoreMesh(core_axis_name="c", subcore_axis_name="s")


@functools.partial(
    pl.kernel,
    out_type=jax.ShapeDtypeStruct((NC, NP), jnp.float32),
    mesh=_mesh,
    scratch_types=[
        pltpu.VMEM((NCHUNK, CH), jnp.int32),
        pltpu.VMEM((CH,), jnp.float32),
        pltpu.VMEM_SHARED((NP,), jnp.float32),
        pltpu.SemaphoreType.DMA,
    ],
)
def _deg_sc(col_hbm, ones_hbm, zeros_hbm, out_hbm, colbuf, ones_v, deg_sh, sem):
    cid = lax.axis_index("c")
    sid = lax.axis_index("s")
    wid = sid * NC + cid

    @pl.when(sid == 0)
    def _():
        pltpu.sync_copy(zeros_hbm, deg_sh)

    pltpu.sync_copy(ones_hbm, ones_v)
    pltpu.sync_copy(col_hbm.at[wid], colbuf)
    plsc.subcore_barrier()

    def body(ci, carry):
        pltpu.async_copy(ones_v, deg_sh.at[colbuf.at[ci]], sem, add=True).wait()
        return carry

    lax.fori_loop(0, NCHUNK, body, 0)
    plsc.subcore_barrier()

    @pl.when(sid == 0)
    def _():
        pltpu.sync_copy(deg_sh, out_hbm.at[cid])


@functools.partial(
    pl.kernel,
    out_type=jax.ShapeDtypeStruct((NC, AGGH, H), jnp.float32),
    mesh=_mesh,
    scratch_types=[
        pltpu.VMEM((4, 2, CH2), jnp.int32),
        pltpu.VMEM((2, CH2, H), jnp.float32),
        pltpu.VMEM_SHARED((HSR, H), jnp.float32),
        pltpu.VMEM_SHARED((AGGH, H), jnp.float32),
        pltpu.SemaphoreType.DMA,
        pltpu.SemaphoreType.DMA,
        pltpu.SemaphoreType.DMA,
        pltpu.SemaphoreType.DMA,
        pltpu.SemaphoreType.DMA,
        pltpu.SemaphoreType.DMA,
        pltpu.SemaphoreType.DMA,
        pltpu.SemaphoreType.DMA,
    ],
)
def _agg_sc(hs_hbm, rc_hbm, zeros_hbm, out_hbm,
            idx, rows, hs_sh, agg_sh, i0, i1, i2, i3, g0, g1, s0, s1):
    cid = lax.axis_index("c")
    sid = lax.axis_index("s")
    isems = (i0, i1, i2, i3)
    gsems = (g0, g1)
    ssems = (s0, s1)
    base = cid * HALF
    iota16 = lax.broadcasted_iota(jnp.int32, (16,), 0)

    def idx_start(c, s4):
        pltpu.async_copy(rc_hbm.at[sid, c], idx.at[s4], isems[s4])

    def idx_wait(s4):
        pltpu.make_async_copy(rc_hbm.at[sid, 0], idx.at[s4],
                              isems[s4]).wait()

    def remap(s4):
        for g in range(CH2 // 16):
            cv = idx[s4, 1, pl.ds(g * 16, 16)]
            l = cv - base
            m = (l >= 0) & (l < HALF)
            idx[s4, 1, pl.ds(g * 16, 16)] = jnp.where(m, l, HALF + iota16)

    def gather_start(s4, s2):
        pltpu.async_copy(hs_sh.at[idx.at[s4, 0]], rows.at[s2], gsems[s2])

    def gather_wait(s4, s2):
        pltpu.make_async_copy(hs_sh.at[idx.at[s4, 0]], rows.at[s2],
                              gsems[s2]).wait()

    def scatter_start(s4, s2):
        pltpu.async_copy(rows.at[s2], agg_sh.at[idx.at[s4, 1]], ssems[s2],
                         add=True)

    def scatter_wait(s4, s2):
        pltpu.make_async_copy(rows.at[s2], agg_sh.at[idx.at[s4, 1]],
                              ssems[s2]).wait()

    @pl.when(sid < NS - 1)
    def _():
        pltpu.sync_copy(hs_hbm.at[pl.ds(sid * RPS, RPS)],
                        hs_sh.at[pl.ds(sid * RPS, RPS)])

    @pl.when(sid == NS - 1)
    def _():
        pltpu.sync_copy(hs_hbm.at[pl.ds((NS - 1) * RPS, HSR - (NS - 1) * RPS)],
                        hs_sh.at[pl.ds((NS - 1) * RPS, HSR - (NS - 1) * RPS)])

    pltpu.sync_copy(zeros_hbm.at[pl.ds(0, RPSH)],
                    agg_sh.at[pl.ds(sid * RPSH, RPSH)])
    idx_start(0, 0)
    idx_start(1, 1)
    plsc.subcore_barrier()
    idx_wait(0)
    remap(0)
    gather_start(0, 0)

    def step(c, k):
        s4 = k
        s2 = k & 1

        @pl.when(c >= 1)
        def _():
            scatter_wait((k - 1) & 3, (k - 1) & 1)

        @pl.when(c + 1 < NCH2)
        def _():
            idx_wait((k + 1) & 3)
            remap((k + 1) & 3)
            gather_start((k + 1) & 3, (k + 1) & 1)

        gather_wait(s4, s2)
        scatter_start(s4, s2)

        @pl.when(c + 2 < NCH2)
        def _():
            idx_start(c + 2, (k + 2) & 3)

    def body(q, carry):
        for k in range(4):
            step(4 * q + k, k)
        return carry

    lax.fori_loop(0, NCH2 // 4, body, 0)
    scatter_wait(3, 1)
    plsc.subcore_barrier()

    pltpu.sync_copy(agg_sh.at[pl.ds(sid * RPSH, RPSH)],
                    out_hbm.at[cid, pl.ds(sid * RPSH, RPSH)])


def _prep_body(deg_ref, x_ref, w_ref, dinv_ref, hs_ref):
    deg = deg_ref[0] + deg_ref[1] + 1.0
    dinv = lax.rsqrt(deg)[:, None]
    dinv_ref[...] = dinv
    hs_ref[...] = jnp.dot(x_ref[...], w_ref[...],
                          preferred_element_type=jnp.float32) * dinv


_prep_tc = pl.pallas_call(
    _prep_body,
    out_shape=(jax.ShapeDtypeStruct((NP, 1), jnp.float32),
               jax.ShapeDtypeStruct((NP, H), jnp.float32)),
)


def _mid_body(agg_ref, hs_ref, dinv_ref, b_ref, w_ref, out_ref):
    dinv = dinv_ref[...]
    agg = jnp.concatenate([agg_ref[0, :HALF], agg_ref[1, :HALF]], axis=0)
    t = (agg + hs_ref[...]) * dinv + b_ref[...]
    t = jnp.maximum(t, 0.0)
    out_ref[...] = jnp.dot(t, w_ref[...],
                           preferred_element_type=jnp.float32) * dinv


_mid_tc = pl.pallas_call(
    _mid_body,
    out_shape=jax.ShapeDtypeStruct((NP, H), jnp.float32),
)


def _final_body(agg_ref, hs_ref, dinv_ref, b_ref, batch_ref, l1w_ref, l1b_ref,
                gam_ref, beta_ref, l2w_ref, l2b_ref, out_ref):
    agg = jnp.concatenate([agg_ref[0, :HALF], agg_ref[1, :HALF]], axis=0)
    h3 = (agg + hs_ref[...]) * dinv_ref[...] + b_ref[...]
    onehot = (lax.broadcasted_iota(jnp.int32, (G, NP), 0)
              == batch_ref[...]).astype(jnp.float32)
    gp = jnp.dot(onehot, h3, preferred_element_type=jnp.float32,
                 precision=lax.Precision.HIGHEST)
    z = jnp.dot(gp, l1w_ref[...], preferred_element_type=jnp.float32)
    z = jnp.maximum(z + l1b_ref[...], 0.0)
    mean = jnp.mean(z, axis=0, keepdims=True)
    var = jnp.mean((z - mean) ** 2, axis=0, keepdims=True)
    z = (z - mean) / jnp.sqrt(var + 1e-5) * gam_ref[...] + beta_ref[...]
    out_ref[...] = jnp.dot(z, l2w_ref[...],
                           preferred_element_type=jnp.float32) + l2b_ref[...]


_final_tc = pl.pallas_call(
    _final_body,
    out_shape=jax.ShapeDtypeStruct((G, 1), jnp.float32),
)


def kernel(x, edge_index, batch, W0, b0, W1, b1, W2, b2,
           lin1_W, lin1_b, bn_gamma, bn_beta, lin2_W, lin2_b):
    pad_e = EP - E
    rowp = jnp.concatenate(
        [edge_index[0], jnp.zeros((pad_e,), jnp.int32)]).reshape(NW, NCHUNK, CH)
    colp = jnp.concatenate(
        [edge_index[1], jnp.full((pad_e,), N, jnp.int32)]).reshape(NW, NCHUNK, CH)
    rc = jnp.stack([rowp.reshape(NS, NCH2, CH2),
                    colp.reshape(NS, NCH2, CH2)], axis=2)
    xp = jnp.pad(x, ((0, NP - N), (0, 0)))
    batchp = jnp.pad(batch, (0, NP - N), constant_values=G).reshape(1, NP)
    zeros2d = jnp.zeros((NP, H), jnp.float32)
    zeros1d = jnp.zeros((NP,), jnp.float32)
    ones_ch = jnp.ones((CH,), jnp.float32)

    deg2 = _deg_sc(colp, ones_ch, zeros1d)
    dinv, hs = _prep_tc(deg2, xp, W0)
    agg = _agg_sc(hs, rc, zeros2d)
    hs = _mid_tc(agg, hs, dinv, b0.reshape(1, H), W1)
    agg = _agg_sc(hs, rc, zeros2d)
    hs = _mid_tc(agg, hs, dinv, b1.reshape(1, H), W2)
    agg = _agg_sc(hs, rc, zeros2d)
    return _final_tc(agg, hs, dinv, b2.reshape(1, H), batchp,
                     lin1_W, lin1_b.reshape(1, 64),
                     bn_gamma.reshape(1, 64), bn_beta.reshape(1, 64),
                     lin2_W, lin2_b.reshape(1, 1))


# speedup vs baseline: 1.1602x; 1.0006x over previous
import functools

import jax
import jax.numpy as jnp
from jax import lax
from jax.experimental import pallas as pl
from jax.experimental.pallas import tpu as pltpu
from jax.experimental.pallas import tpu_sc as plsc

N = 10000
E = 320000
D = 128
H = 128
G = 64

NC = 2
NS = 16
NW = NC * NS

NP = 10112
CH = 128
EPW = 10240
NCHUNK = EPW // CH
EP = EPW * NW
RPS = NP // NS

HALF = NP // 2
AGGH = 5120
RPSH = AGGH // NS
CH2 = 32
EPT = EP // NS
NCH2 = EPT // CH2
HSR = 10048

_mesh = plsc.VectorSubcoreMesh(core_axis_name="c", subcore_axis_name="s")


@functools.partial(
    pl.kernel,
    out_type=jax.ShapeDtypeStruct((NC, NP), jnp.float32),
    mesh=_mesh,
    scratch_types=[
        pltpu.VMEM((NCHUNK, CH), jnp.int32),
        pltpu.VMEM((CH,), jnp.float32),
        pltpu.VMEM_SHARED((NP,), jnp.float32),
        pltpu.SemaphoreType.DMA,
    ],
)
def _deg_sc(col_hbm, ones_hbm, zeros_hbm, out_hbm, colbuf, ones_v, deg_sh, sem):
    cid = lax.axis_index("c")
    sid = lax.axis_index("s")
    wid = sid * NC + cid

    @pl.when(sid == 0)
    def _():
        pltpu.sync_copy(zeros_hbm, deg_sh)

    pltpu.sync_copy(ones_hbm, ones_v)
    pltpu.sync_copy(col_hbm.at[wid], colbuf)
    plsc.subcore_barrier()

    def body(ci, carry):
        pltpu.async_copy(ones_v, deg_sh.at[colbuf.at[ci]], sem, add=True).wait()
        return carry

    lax.fori_loop(0, NCHUNK, body, 0)
    plsc.subcore_barrier()

    @pl.when(sid == 0)
    def _():
        pltpu.sync_copy(deg_sh, out_hbm.at[cid])


@functools.partial(
    pl.kernel,
    out_type=jax.ShapeDtypeStruct((NC, AGGH, H), jnp.float32),
    mesh=_mesh,
    scratch_types=[
        pltpu.VMEM((4, 2, CH2), jnp.int32),
        pltpu.VMEM((2, CH2, H), jnp.float32),
        pltpu.VMEM_SHARED((HSR, H), jnp.float32),
        pltpu.VMEM_SHARED((AGGH, H), jnp.float32),
        pltpu.SemaphoreType.DMA,
        pltpu.SemaphoreType.DMA,
        pltpu.SemaphoreType.DMA,
        pltpu.SemaphoreType.DMA,
        pltpu.SemaphoreType.DMA,
        pltpu.SemaphoreType.DMA,
        pltpu.SemaphoreType.DMA,
        pltpu.SemaphoreType.DMA,
    ],
)
def _agg_sc(hs_hbm, rc_hbm, zeros_hbm, out_hbm,
            idx, rows, hs_sh, agg_sh, i0, i1, i2, i3, g0, g1, s0, s1):
    cid = lax.axis_index("c")
    sid = lax.axis_index("s")
    isems = (i0, i1, i2, i3)
    gsems = (g0, g1)
    ssems = (s0, s1)
    base = cid * HALF
    iota16 = lax.broadcasted_iota(jnp.int32, (16,), 0)

    def idx_start(c, s4):
        pltpu.async_copy(rc_hbm.at[sid, c], idx.at[s4], isems[s4])

    def idx_wait(s4):
        pltpu.make_async_copy(rc_hbm.at[sid, 0], idx.at[s4],
                              isems[s4]).wait()

    def remap(s4):
        for g in range(CH2 // 16):
            cv = idx[s4, 1, pl.ds(g * 16, 16)]
            l = cv - base
            m = (l >= 0) & (l < HALF)
            idx[s4, 1, pl.ds(g * 16, 16)] = jnp.where(m, l, HALF + iota16)

    def gather_start(s4, s2):
        pltpu.async_copy(hs_sh.at[idx.at[s4, 0]], rows.at[s2], gsems[s2])

    def gather_wait(s4, s2):
        pltpu.make_async_copy(hs_sh.at[idx.at[s4, 0]], rows.at[s2],
                              gsems[s2]).wait()

    def scatter_start(s4, s2):
        pltpu.async_copy(rows.at[s2], agg_sh.at[idx.at[s4, 1]], ssems[s2],
                         add=True)

    def scatter_wait(s4, s2):
        pltpu.make_async_copy(rows.at[s2], agg_sh.at[idx.at[s4, 1]],
                              ssems[s2]).wait()

    @pl.when(sid < NS - 1)
    def _():
        pltpu.sync_copy(hs_hbm.at[pl.ds(sid * RPS, RPS)],
                        hs_sh.at[pl.ds(sid * RPS, RPS)])

    @pl.when(sid == NS - 1)
    def _():
        pltpu.sync_copy(hs_hbm.at[pl.ds((NS - 1) * RPS, HSR - (NS - 1) * RPS)],
                        hs_sh.at[pl.ds((NS - 1) * RPS, HSR - (NS - 1) * RPS)])

    pltpu.sync_copy(zeros_hbm.at[pl.ds(0, RPSH)],
                    agg_sh.at[pl.ds(sid * RPSH, RPSH)])
    idx_start(0, 0)
    idx_start(1, 1)
    plsc.subcore_barrier()
    idx_wait(0)
    remap(0)
    gather_start(0, 0)

    def step(c, k, first, last):
        s4 = k
        s2 = k & 1
        if not (first and k == 0):
            scatter_wait((k - 1) & 3, (k - 1) & 1)
        if not (last and k == 3):
            idx_wait((k + 1) & 3)
            remap((k + 1) & 3)
            gather_start((k + 1) & 3, (k + 1) & 1)
        gather_wait(s4, s2)
        scatter_start(s4, s2)
        if not (last and k >= 2):
            idx_start(c + 2, (k + 2) & 3)

    for k in range(4):
        step(k, k, True, False)

    def body(q, carry):
        for k in range(4):
            step(4 * q + k, k, False, False)
        return carry

    lax.fori_loop(1, NCH2 // 4 - 1, body, 0)
    for k in range(4):
        step(NCH2 - 4 + k, k, False, True)
    scatter_wait(3, 1)
    plsc.subcore_barrier()

    pltpu.sync_copy(agg_sh.at[pl.ds(sid * RPSH, RPSH)],
                    out_hbm.at[cid, pl.ds(sid * RPSH, RPSH)])


def _prep_body(deg_ref, x_ref, w_ref, dinv_ref, hs_ref):
    deg = deg_ref[0] + deg_ref[1] + 1.0
    dinv = lax.rsqrt(deg)[:, None]
    dinv_ref[...] = dinv
    hs_ref[...] = jnp.dot(x_ref[...], w_ref[...],
                          preferred_element_type=jnp.float32) * dinv


_prep_tc = pl.pallas_call(
    _prep_body,
    out_shape=(jax.ShapeDtypeStruct((NP, 1), jnp.float32),
               jax.ShapeDtypeStruct((NP, H), jnp.float32)),
)


def _mid_body(agg_ref, hs_ref, dinv_ref, b_ref, w_ref, out_ref):
    dinv = dinv_ref[...]
    agg = jnp.concatenate([agg_ref[0, :HALF], agg_ref[1, :HALF]], axis=0)
    t = (agg + hs_ref[...]) * dinv + b_ref[...]
    t = jnp.maximum(t, 0.0)
    out_ref[...] = jnp.dot(t, w_ref[...],
                           preferred_element_type=jnp.float32) * dinv


_mid_tc = pl.pallas_call(
    _mid_body,
    out_shape=jax.ShapeDtypeStruct((NP, H), jnp.float32),
)


def _final_body(agg_ref, hs_ref, dinv_ref, b_ref, batch_ref, l1w_ref, l1b_ref,
                gam_ref, beta_ref, l2w_ref, l2b_ref, out_ref):
    agg = jnp.concatenate([agg_ref[0, :HALF], agg_ref[1, :HALF]], axis=0)
    h3 = (agg + hs_ref[...]) * dinv_ref[...] + b_ref[...]
    onehot = (lax.broadcasted_iota(jnp.int32, (G, NP), 0)
              == batch_ref[...]).astype(jnp.float32)
    gp = jnp.dot(onehot, h3, preferred_element_type=jnp.float32,
                 precision=lax.Precision.HIGHEST)
    z = jnp.dot(gp, l1w_ref[...], preferred_element_type=jnp.float32)
    z = jnp.maximum(z + l1b_ref[...], 0.0)
    mean = jnp.mean(z, axis=0, keepdims=True)
    var = jnp.mean((z - mean) ** 2, axis=0, keepdims=True)
    z = (z - mean) / jnp.sqrt(var + 1e-5) * gam_ref[...] + beta_ref[...]
    out_ref[...] = jnp.dot(z, l2w_ref[...],
                           preferred_element_type=jnp.float32) + l2b_ref[...]


_final_tc = pl.pallas_call(
    _final_body,
    out_shape=jax.ShapeDtypeStruct((G, 1), jnp.float32),
)


def kernel(x, edge_index, batch, W0, b0, W1, b1, W2, b2,
           lin1_W, lin1_b, bn_gamma, bn_beta, lin2_W, lin2_b):
    pad_e = EP - E
    rowp = jnp.concatenate(
        [edge_index[0], jnp.zeros((pad_e,), jnp.int32)]).reshape(NW, NCHUNK, CH)
    colp = jnp.concatenate(
        [edge_index[1], jnp.full((pad_e,), N, jnp.int32)]).reshape(NW, NCHUNK, CH)
    rc = jnp.stack([rowp.reshape(NS, NCH2, CH2),
                    colp.reshape(NS, NCH2, CH2)], axis=2)
    xp = jnp.pad(x, ((0, NP - N), (0, 0)))
    batchp = jnp.pad(batch, (0, NP - N), constant_values=G).reshape(1, NP)
    zeros2d = jnp.zeros((NP, H), jnp.float32)
    zeros1d = jnp.zeros((NP,), jnp.float32)
    ones_ch = jnp.ones((CH,), jnp.float32)

    deg2 = _deg_sc(colp, ones_ch, zeros1d)
    dinv, hs = _prep_tc(deg2, xp, W0)
    agg = _agg_sc(hs, rc, zeros2d)
    hs = _mid_tc(agg, hs, dinv, b0.reshape(1, H), W1)
    agg = _agg_sc(hs, rc, zeros2d)
    hs = _mid_tc(agg, hs, dinv, b1.reshape(1, H), W2)
    agg = _agg_sc(hs, rc, zeros2d)
    return _final_tc(agg, hs, dinv, b2.reshape(1, H), batchp,
                     lin1_W, lin1_b.reshape(1, 64),
                     bn_gamma.reshape(1, 64), bn_beta.reshape(1, 64),
                     lin2_W, lin2_b.reshape(1, 1))


# speedup vs baseline: 1.1605x; 1.0002x over previous
import functools

import jax
import jax.numpy as jnp
from jax import lax
from jax.experimental import pallas as pl
from jax.experimental.pallas import tpu as pltpu
from jax.experimental.pallas import tpu_sc as plsc

N = 10000
E = 320000
D = 128
H = 128
G = 64

NC = 2
NS = 16
NW = NC * NS

NP = 10112
CH = 128
EPW = 10240
NCHUNK = EPW // CH
EP = EPW * NW
RPS = NP // NS

HALF = NP // 2
AGGH = 5120
RPSH = AGGH // NS
CH2 = 32
EPT = EP // NS
NCH2 = EPT // CH2
HSR = 10048

_mesh = plsc.VectorSubcoreMesh(core_axis_name="c", subcore_axis_name="s")


@functools.partial(
    pl.kernel,
    out_type=jax.ShapeDtypeStruct((NC, NP), jnp.float32),
    mesh=_mesh,
    scratch_types=[
        pltpu.VMEM((NCHUNK, CH), jnp.int32),
        pltpu.VMEM((CH,), jnp.float32),
        pltpu.VMEM_SHARED((NP,), jnp.float32),
        pltpu.SemaphoreType.DMA,
    ],
)
def _deg_sc(col_hbm, ones_hbm, zeros_hbm, out_hbm, colbuf, ones_v, deg_sh, sem):
    cid = lax.axis_index("c")
    sid = lax.axis_index("s")
    wid = sid * NC + cid

    @pl.when(sid == 0)
    def _():
        pltpu.sync_copy(zeros_hbm, deg_sh)

    pltpu.sync_copy(ones_hbm, ones_v)
    pltpu.sync_copy(col_hbm.at[wid], colbuf)
    plsc.subcore_barrier()

    def body(ci, carry):
        pltpu.async_copy(ones_v, deg_sh.at[colbuf.at[ci]], sem, add=True).wait()
        return carry

    lax.fori_loop(0, NCHUNK, body, 0)
    plsc.subcore_barrier()

    @pl.when(sid == 0)
    def _():
        pltpu.sync_copy(deg_sh, out_hbm.at[cid])


@functools.partial(
    pl.kernel,
    out_type=jax.ShapeDtypeStruct((NC, AGGH, H), jnp.float32),
    mesh=_mesh,
    scratch_types=[
        pltpu.VMEM((4, 2, CH2), jnp.int32),
        pltpu.VMEM((2, CH2, H), jnp.float32),
        pltpu.VMEM_SHARED((HSR, H), jnp.float32),
        pltpu.VMEM_SHARED((AGGH, H), jnp.float32),
        pltpu.SemaphoreType.DMA,
        pltpu.SemaphoreType.DMA,
        pltpu.SemaphoreType.DMA,
        pltpu.SemaphoreType.DMA,
        pltpu.SemaphoreType.DMA,
        pltpu.SemaphoreType.DMA,
        pltpu.SemaphoreType.DMA,
        pltpu.SemaphoreType.DMA,
    ],
)
def _agg_sc(hs_hbm, rc_hbm, zeros_hbm, out_hbm,
            idx, rows, hs_sh, agg_sh, i0, i1, i2, i3, g0, g1, s0, s1):
    cid = lax.axis_index("c")
    sid = lax.axis_index("s")
    isems = (i0, i1, i2, i3)
    gsems = (g0, g1)
    ssems = (s0, s1)
    base = cid * HALF
    iota16 = lax.broadcasted_iota(jnp.int32, (16,), 0)
    disc = HALF + iota16 + (sid % 4) * 16

    def idx_start(c, s4):
        pltpu.async_copy(rc_hbm.at[sid, c], idx.at[s4], isems[s4])

    def idx_wait(s4):
        pltpu.make_async_copy(rc_hbm.at[sid, 0], idx.at[s4],
                              isems[s4]).wait()

    def remap(s4):
        for g in range(CH2 // 16):
            cv = idx[s4, 1, pl.ds(g * 16, 16)]
            l = cv - base
            m = (l >= 0) & (l < HALF)
            idx[s4, 1, pl.ds(g * 16, 16)] = jnp.where(m, l, disc)

    def gather_start(s4, s2):
        pltpu.async_copy(hs_sh.at[idx.at[s4, 0]], rows.at[s2], gsems[s2])

    def gather_wait(s4, s2):
        pltpu.make_async_copy(hs_sh.at[idx.at[s4, 0]], rows.at[s2],
                              gsems[s2]).wait()

    def scatter_start(s4, s2):
        pltpu.async_copy(rows.at[s2], agg_sh.at[idx.at[s4, 1]], ssems[s2],
                         add=True)

    def scatter_wait(s4, s2):
        pltpu.make_async_copy(rows.at[s2], agg_sh.at[idx.at[s4, 1]],
                              ssems[s2]).wait()

    @pl.when(sid < NS - 1)
    def _():
        pltpu.sync_copy(hs_hbm.at[pl.ds(sid * RPS, RPS)],
                        hs_sh.at[pl.ds(sid * RPS, RPS)])

    @pl.when(sid == NS - 1)
    def _():
        pltpu.sync_copy(hs_hbm.at[pl.ds((NS - 1) * RPS, HSR - (NS - 1) * RPS)],
                        hs_sh.at[pl.ds((NS - 1) * RPS, HSR - (NS - 1) * RPS)])

    pltpu.sync_copy(zeros_hbm.at[pl.ds(0, RPSH)],
                    agg_sh.at[pl.ds(sid * RPSH, RPSH)])
    idx_start(0, 0)
    idx_start(1, 1)
    plsc.subcore_barrier()
    idx_wait(0)
    remap(0)
    gather_start(0, 0)

    def step(c, k, first, last):
        s4 = k
        s2 = k & 1
        if not (first and k == 0):
            scatter_wait((k - 1) & 3, (k - 1) & 1)
        if not (last and k == 3):
            idx_wait((k + 1) & 3)
            remap((k + 1) & 3)
            gather_start((k + 1) & 3, (k + 1) & 1)
        gather_wait(s4, s2)
        scatter_start(s4, s2)
        if not (last and k >= 2):
            idx_start(c + 2, (k + 2) & 3)

    for k in range(4):
        step(k, k, True, False)

    def body(q, carry):
        for k in range(4):
            step(4 * q + k, k, False, False)
        return carry

    lax.fori_loop(1, NCH2 // 4 - 1, body, 0)
    for k in range(4):
        step(NCH2 - 4 + k, k, False, True)
    scatter_wait(3, 1)
    plsc.subcore_barrier()

    pltpu.sync_copy(agg_sh.at[pl.ds(sid * RPSH, RPSH)],
                    out_hbm.at[cid, pl.ds(sid * RPSH, RPSH)])


def _prep_body(deg_ref, x_ref, w_ref, dinv_ref, hs_ref):
    deg = deg_ref[0] + deg_ref[1] + 1.0
    dinv = lax.rsqrt(deg)[:, None]
    dinv_ref[...] = dinv
    hs_ref[...] = jnp.dot(x_ref[...], w_ref[...],
                          preferred_element_type=jnp.float32) * dinv


_prep_tc = pl.pallas_call(
    _prep_body,
    out_shape=(jax.ShapeDtypeStruct((NP, 1), jnp.float32),
               jax.ShapeDtypeStruct((NP, H), jnp.float32)),
)


def _mid_body(agg_ref, hs_ref, dinv_ref, b_ref, w_ref, out_ref):
    dinv = dinv_ref[...]
    agg = jnp.concatenate([agg_ref[0, :HALF], agg_ref[1, :HALF]], axis=0)
    t = (agg + hs_ref[...]) * dinv + b_ref[...]
    t = jnp.maximum(t, 0.0)
    out_ref[...] = jnp.dot(t, w_ref[...],
                           preferred_element_type=jnp.float32) * dinv


_mid_tc = pl.pallas_call(
    _mid_body,
    out_shape=jax.ShapeDtypeStruct((NP, H), jnp.float32),
)


def _final_body(agg_ref, hs_ref, dinv_ref, b_ref, batch_ref, l1w_ref, l1b_ref,
                gam_ref, beta_ref, l2w_ref, l2b_ref, out_ref):
    agg = jnp.concatenate([agg_ref[0, :HALF], agg_ref[1, :HALF]], axis=0)
    h3 = (agg + hs_ref[...]) * dinv_ref[...] + b_ref[...]
    onehot = (lax.broadcasted_iota(jnp.int32, (G, NP), 0)
              == batch_ref[...]).astype(jnp.float32)
    gp = jnp.dot(onehot, h3, preferred_element_type=jnp.float32,
                 precision=lax.Precision.HIGHEST)
    z = jnp.dot(gp, l1w_ref[...], preferred_element_type=jnp.float32)
    z = jnp.maximum(z + l1b_ref[...], 0.0)
    mean = jnp.mean(z, axis=0, keepdims=True)
    var = jnp.mean((z - mean) ** 2, axis=0, keepdims=True)
    z = (z - mean) / jnp.sqrt(var + 1e-5) * gam_ref[...] + beta_ref[...]
    out_ref[...] = jnp.dot(z, l2w_ref[...],
                           preferred_element_type=jnp.float32) + l2b_ref[...]


_final_tc = pl.pallas_call(
    _final_body,
    out_shape=jax.ShapeDtypeStruct((G, 1), jnp.float32),
)


def kernel(x, edge_index, batch, W0, b0, W1, b1, W2, b2,
           lin1_W, lin1_b, bn_gamma, bn_beta, lin2_W, lin2_b):
    pad_e = EP - E
    rowp = jnp.concatenate(
        [edge_index[0], jnp.zeros((pad_e,), jnp.int32)]).reshape(NW, NCHUNK, CH)
    colp = jnp.concatenate(
        [edge_index[1], jnp.full((pad_e,), N, jnp.int32)]).reshape(NW, NCHUNK, CH)
    rc = jnp.stack([rowp.reshape(NS, NCH2, CH2),
                    colp.reshape(NS, NCH2, CH2)], axis=2)
    xp = jnp.pad(x, ((0, NP - N), (0, 0)))
    batchp = jnp.pad(batch, (0, NP - N), constant_values=G).reshape(1, NP)
    zeros2d = jnp.zeros((NP, H), jnp.float32)
    zeros1d = jnp.zeros((NP,), jnp.float32)
    ones_ch = jnp.ones((CH,), jnp.float32)

    deg2 = _deg_sc(colp, ones_ch, zeros1d)
    dinv, hs = _prep_tc(deg2, xp, W0)
    agg = _agg_sc(hs, rc, zeros2d)
    hs = _mid_tc(agg, hs, dinv, b0.reshape(1, H), W1)
    agg = _agg_sc(hs, rc, zeros2d)
    hs = _mid_tc(agg, hs, dinv, b1.reshape(1, H), W2)
    agg = _agg_sc(hs, rc, zeros2d)
    return _final_tc(agg, hs, dinv, b2.reshape(1, H), batchp,
                     lin1_W, lin1_b.reshape(1, 64),
                     bn_gamma.reshape(1, 64), bn_beta.reshape(1, 64),
                     lin2_W, lin2_b.reshape(1, 1))
